# Initial kernel scaffold; baseline (speedup 1.0000x reference)
import jax
import jax.numpy as jnp
from jax.experimental import pallas as pl


def kernel(x, species, factors, scale_params, shift_params):
    raise NotImplementedError("write your pallas kernel here")


# speedup vs baseline: 1.5186x; 1.5186x over previous
import functools

import jax
import jax.numpy as jnp
from jax import lax
from jax.experimental import pallas as pl
from jax.experimental.pallas import tpu as pltpu
from jax.experimental.pallas import tpu_sc as plsc

N_ATOMS = 100000
N_SPECIES = 16
NC, NS, L = 2, 16, 16
NW = NC * NS
NPAD = 100352
CHUNK = NPAD // NW


def _sc_body(x_hbm, sp_hbm, tab_hbm, out_hbm,
             x_v, sp_v, o_v, tab_v, a_v, b_v, sem_x, sem_sp, sem_t):
    wid = lax.axis_index("s") * NC + lax.axis_index("c")
    base = wid * CHUNK

    cp_x = pltpu.async_copy(x_hbm.at[pl.ds(base, CHUNK)], x_v, sem_x)
    cp_sp = pltpu.async_copy(sp_hbm.at[pl.ds(base, CHUNK)], sp_v, sem_sp)
    cp_t = pltpu.async_copy(tab_hbm, tab_v, sem_t)

    cp_t.wait()
    f = tab_v[pl.ds(0, L)]
    a_v[...] = f * tab_v[pl.ds(L, L)]
    b_v[...] = f * tab_v[pl.ds(2 * L, L)]
    cp_x.wait()
    cp_sp.wait()

    @plsc.parallel_loop(0, CHUNK, step=L)
    def _(i):
        sp = sp_v[pl.ds(i, L)]
        xv = x_v[pl.ds(i, L)]
        av = plsc.load_gather(a_v, [sp])
        bv = plsc.load_gather(b_v, [sp])
        o_v[pl.ds(i, L)] = av * xv + bv

    pltpu.sync_copy(o_v, out_hbm.at[pl.ds(base, CHUNK)])


_sc_call = pl.kernel(
    _sc_body,
    out_type=jax.ShapeDtypeStruct((NPAD,), jnp.float32),
    mesh=plsc.VectorSubcoreMesh(
        core_axis_name="c", subcore_axis_name="s",
        num_cores=NC, num_subcores=NS),
    compiler_params=pltpu.CompilerParams(needs_layout_passes=False),
    scratch_types=[
        pltpu.VMEM((CHUNK,), jnp.float32),
        pltpu.VMEM((CHUNK,), jnp.int32),
        pltpu.VMEM((CHUNK,), jnp.float32),
        pltpu.VMEM((3 * L,), jnp.float32),
        pltpu.VMEM((L,), jnp.float32),
        pltpu.VMEM((L,), jnp.float32),
        pltpu.SemaphoreType.DMA,
        pltpu.SemaphoreType.DMA,
        pltpu.SemaphoreType.DMA,
    ],
)


@jax.jit
def kernel(x, species, factors, scale_params, shift_params):
    xp = jnp.pad(x.reshape(-1), (0, NPAD - N_ATOMS))
    spp = jnp.pad(species, (0, NPAD - N_ATOMS))
    tab = jnp.concatenate([factors, scale_params, shift_params])
    out = _sc_call(xp, spp, tab)
    return out[:N_ATOMS].reshape(N_ATOMS, 1)


# speedup vs baseline: 1.6225x; 1.0684x over previous
import jax
import jax.numpy as jnp
from jax import lax
from jax.experimental import pallas as pl
from jax.experimental.pallas import tpu as pltpu
from jax.experimental.pallas import tpu_sc as plsc

N_ATOMS = 100000
N_SPECIES = 16
NC, NS, L = 2, 16, 16
NW = NC * NS
CHUNK = 3120
LAST = N_ATOMS - (NW - 1) * CHUNK


def _sc_body(x_hbm, sp_hbm, fac_hbm, scl_hbm, shf_hbm, out_hbm,
             x_v, sp_v, o_v, tabs_v, a_v, b_v, sem_x, sem_sp, sem_t):
    wid = lax.axis_index("s") * NC + lax.axis_index("c")
    base = wid * CHUNK

    def work(n):
        cp_x = pltpu.async_copy(x_hbm.at[pl.ds(base, n)], x_v.at[pl.ds(0, n)],
                                sem_x)
        cp_sp = pltpu.async_copy(sp_hbm.at[pl.ds(base, n)],
                                 sp_v.at[pl.ds(0, n)], sem_sp)
        cp_f = pltpu.async_copy(fac_hbm, tabs_v.at[0], sem_t)
        cp_s = pltpu.async_copy(scl_hbm, tabs_v.at[1], sem_t)
        cp_h = pltpu.async_copy(shf_hbm, tabs_v.at[2], sem_t)

        cp_f.wait()
        cp_s.wait()
        cp_h.wait()
        f = tabs_v[0, :]
        a_v[...] = f * tabs_v[1, :]
        b_v[...] = f * tabs_v[2, :]
        cp_x.wait()
        cp_sp.wait()

        @plsc.parallel_loop(0, n, step=L)
        def _(i):
            sp = sp_v[pl.ds(i, L)]
            xv = x_v[pl.ds(i, L)]
            av = plsc.load_gather(a_v, [sp])
            bv = plsc.load_gather(b_v, [sp])
            o_v[pl.ds(i, L)] = av * xv + bv

        pltpu.sync_copy(o_v.at[pl.ds(0, n)], out_hbm.at[pl.ds(base, n)])

    @pl.when(wid != NW - 1)
    def _():
        work(CHUNK)

    @pl.when(wid == NW - 1)
    def _():
        work(LAST)


_sc_call = pl.kernel(
    _sc_body,
    out_type=jax.ShapeDtypeStruct((N_ATOMS,), jnp.float32),
    mesh=plsc.VectorSubcoreMesh(
        core_axis_name="c", subcore_axis_name="s",
        num_cores=NC, num_subcores=NS),
    compiler_params=pltpu.CompilerParams(needs_layout_passes=False),
    scratch_types=[
        pltpu.VMEM((LAST,), jnp.float32),
        pltpu.VMEM((LAST,), jnp.int32),
        pltpu.VMEM((LAST,), jnp.float32),
        pltpu.VMEM((3, L), jnp.float32),
        pltpu.VMEM((L,), jnp.float32),
        pltpu.VMEM((L,), jnp.float32),
        pltpu.SemaphoreType.DMA,
        pltpu.SemaphoreType.DMA,
        pltpu.SemaphoreType.DMA,
    ],
)


@jax.jit
def kernel(x, species, factors, scale_params, shift_params):
    out = _sc_call(x.reshape(-1), species, factors, scale_params, shift_params)
    return out.reshape(N_ATOMS, 1)


# speedup vs baseline: 1.6476x; 1.0154x over previous
import jax
import jax.numpy as jnp
from jax import lax
from jax.experimental import pallas as pl
from jax.experimental.pallas import tpu as pltpu
from jax.experimental.pallas import tpu_sc as plsc

N_ATOMS = 100000
N_SPECIES = 16
NC, NS, L = 2, 16, 16
NW = NC * NS
CHUNK = 3120
LAST = N_ATOMS - (NW - 1) * CHUNK


def _sc_body(sp_hbm, fac_hbm, scl_hbm, shf_hbm, oa_hbm, ob_hbm,
             sp_v, oa_v, ob_v, tabs_v, a_v, b_v, sem_sp, sem_t, sem_o):
    wid = lax.axis_index("s") * NC + lax.axis_index("c")
    base = wid * CHUNK
    is_last = wid == NW - 1

    def copy_in(n):
        return pltpu.make_async_copy(sp_hbm.at[pl.ds(base, n)],
                                     sp_v.at[pl.ds(0, n)], sem_sp)

    cp_f = pltpu.async_copy(fac_hbm, tabs_v.at[0], sem_t)
    cp_s = pltpu.async_copy(scl_hbm, tabs_v.at[1], sem_t)
    cp_h = pltpu.async_copy(shf_hbm, tabs_v.at[2], sem_t)

    @pl.when(jnp.logical_not(is_last))
    def _():
        copy_in(CHUNK).start()

    @pl.when(is_last)
    def _():
        copy_in(LAST).start()

    cp_f.wait()
    cp_s.wait()
    cp_h.wait()
    f = tabs_v[0, :]
    a_v[...] = f * tabs_v[1, :]
    b_v[...] = f * tabs_v[2, :]

    @pl.when(jnp.logical_not(is_last))
    def _():
        copy_in(CHUNK).wait()

    @pl.when(is_last)
    def _():
        copy_in(LAST).wait()

    n = jnp.where(is_last, LAST, CHUNK)

    @plsc.parallel_loop(0, n, step=L, unroll=8)
    def _(i):
        sp = sp_v[pl.ds(i, L)]
        oa_v[pl.ds(i, L)] = plsc.load_gather(a_v, [sp])
        ob_v[pl.ds(i, L)] = plsc.load_gather(b_v, [sp])

    def copy_out(n):
        return (pltpu.make_async_copy(oa_v.at[pl.ds(0, n)],
                                      oa_hbm.at[pl.ds(base, n)], sem_o),
                pltpu.make_async_copy(ob_v.at[pl.ds(0, n)],
                                      ob_hbm.at[pl.ds(base, n)], sem_o))

    @pl.when(jnp.logical_not(is_last))
    def _():
        ca, cb = copy_out(CHUNK)
        ca.start()
        cb.start()
        ca.wait()
        cb.wait()

    @pl.when(is_last)
    def _():
        ca, cb = copy_out(LAST)
        ca.start()
        cb.start()
        ca.wait()
        cb.wait()


_sc_call = pl.kernel(
    _sc_body,
    out_type=(jax.ShapeDtypeStruct((N_ATOMS,), jnp.float32),
              jax.ShapeDtypeStruct((N_ATOMS,), jnp.float32)),
    mesh=plsc.VectorSubcoreMesh(
        core_axis_name="c", subcore_axis_name="s",
        num_cores=NC, num_subcores=NS),
    compiler_params=pltpu.CompilerParams(needs_layout_passes=False),
    scratch_types=[
        pltpu.VMEM((LAST,), jnp.int32),
        pltpu.VMEM((LAST,), jnp.float32),
        pltpu.VMEM((LAST,), jnp.float32),
        pltpu.VMEM((3, L), jnp.float32),
        pltpu.VMEM((L,), jnp.float32),
        pltpu.VMEM((L,), jnp.float32),
        pltpu.SemaphoreType.DMA,
        pltpu.SemaphoreType.DMA,
        pltpu.SemaphoreType.DMA,
    ],
)


@jax.jit
def kernel(x, species, factors, scale_params, shift_params):
    a_s, b_s = _sc_call(species, factors, scale_params, shift_params)
    return a_s[:, None] * x + b_s[:, None]
